# Initial kernel scaffold; baseline (speedup 1.0000x reference)
import jax
import jax.numpy as jnp
from jax.experimental import pallas as pl


def kernel(x, edge_index, edge_weight, num_nodes, W0, W1, W2):
    raise NotImplementedError("write your pallas kernel here")


# speedup vs baseline: 4.2477x; 4.2477x over previous
import functools

import jax
import jax.numpy as jnp
from jax import lax
from jax.experimental import pallas as pl
from jax.experimental.pallas import tpu as pltpu
from jax.experimental.pallas import tpu_sc as plsc

NC = 2
NS = 16
CHUNK = 80


def _sc_segment_sum(x, src, dst, w, zeros):
    n, d = x.shape
    e = src.shape[0]
    nw = NC * NS
    epw = e // nw
    assert epw * nw == e and epw % CHUNK == 0
    nchunk = epw // CHUNK
    row_blk = 80
    nrow_blk = n // row_blk
    assert nrow_blk * row_blk == n

    mesh = plsc.VectorSubcoreMesh(core_axis_name="c", subcore_axis_name="s")

    @functools.partial(
        pl.kernel,
        out_type=jax.ShapeDtypeStruct((NC, n, d), jnp.float32),
        mesh=mesh,
        scratch_types=[
            pltpu.VMEM_SHARED((n, d), jnp.float32),
            pltpu.VMEM((CHUNK,), jnp.int32),
            pltpu.VMEM((CHUNK,), jnp.int32),
            pltpu.VMEM((CHUNK + 8,), jnp.float32),
            pltpu.VMEM((CHUNK, d), jnp.float32),
            pltpu.SemaphoreType.DMA,
        ],
        compiler_params=pltpu.CompilerParams(needs_layout_passes=False),
    )
    def sc_kernel(x_hbm, src_hbm, dst_hbm, w_hbm, zeros_hbm, part_hbm,
                  agg_sh, src_v, dst_v, w_v, rows_v, sem):
        c = lax.axis_index("c")
        s = lax.axis_index("s")
        wid = c * NS + s

        @pl.loop(s, nrow_blk, step=NS)
        def zero_loop(b):
            r0 = pl.multiple_of(b * row_blk, 8)
            pltpu.sync_copy(zeros_hbm.at[pl.ds(r0, row_blk)],
                            agg_sh.at[pl.ds(r0, row_blk)])
        plsc.subcore_barrier()

        ebase = wid * epw

        @pl.loop(0, nchunk)
        def edge_loop(j):
            off = pl.multiple_of(ebase + j * CHUNK, CHUNK)
            pltpu.sync_copy(src_hbm.at[pl.ds(off, CHUNK)], src_v)
            pltpu.sync_copy(dst_hbm.at[pl.ds(off, CHUNK)], dst_v)
            pltpu.sync_copy(w_hbm.at[pl.ds(off, CHUNK)],
                            w_v.at[pl.ds(8, CHUNK)])
            pltpu.async_copy(x_hbm.at[src_v], rows_v, sem).wait()
            for ei in range(CHUNK):
                wb = plsc.load_gather(
                    w_v, [jnp.full((16,), ei + 8, dtype=jnp.int32)])
                for r in range(d // 16):
                    sl = pl.ds(r * 16, 16)
                    rows_v[ei, sl] = rows_v[ei, sl] * wb
            pltpu.sync_copy(rows_v, agg_sh.at[dst_v], add=True)

        plsc.subcore_barrier()

        @pl.loop(s, nrow_blk, step=NS)
        def flush_loop(b):
            r0 = pl.multiple_of(b * row_blk, 8)
            pltpu.sync_copy(agg_sh.at[pl.ds(r0, row_blk)],
                            part_hbm.at[c, pl.ds(r0, row_blk)])

    return sc_kernel(x, src, dst, w, zeros)


def _tc_matmul_relu(partials, W0, W1, W2):
    _, n, d = partials.shape
    block_rows = 1000

    def body(p_ref, w0_ref, w1_ref, w2_ref, o_ref):
        pblk = p_ref[0] + p_ref[1]
        w = w0_ref[...] + w1_ref[...] + w2_ref[...]
        acc = jnp.dot(pblk, w, preferred_element_type=jnp.float32)
        o_ref[...] = jnp.maximum(acc, 0.0)

    return pl.pallas_call(
        body,
        grid=(n // block_rows,),
        in_specs=[
            pl.BlockSpec((2, block_rows, d), lambda i: (0, i, 0)),
            pl.BlockSpec((d, d), lambda i: (0, 0)),
            pl.BlockSpec((d, d), lambda i: (0, 0)),
            pl.BlockSpec((d, d), lambda i: (0, 0)),
        ],
        out_specs=pl.BlockSpec((block_rows, d), lambda i: (i, 0)),
        out_shape=jax.ShapeDtypeStruct((n, d), jnp.float32),
    )(partials, W0, W1, W2)


def kernel(x, edge_index, edge_weight, num_nodes, W0, W1, W2):
    src = edge_index[0]
    dst = edge_index[1]
    zeros = jnp.zeros(x.shape, dtype=jnp.float32)
    partials = _sc_segment_sum(x, src, dst, edge_weight, zeros)
    return _tc_matmul_relu(partials, W0, W1, W2)


# speedup vs baseline: 6.5299x; 1.5373x over previous
import functools

import jax
import jax.numpy as jnp
from jax import lax
from jax.experimental import pallas as pl
from jax.experimental.pallas import tpu as pltpu
from jax.experimental.pallas import tpu_sc as plsc

NC = 2
NS = 16
CHUNK = 80
NBUF = 2


def _sc_segment_sum(x, packed3, w, zeros):
    n, d = x.shape
    nw = NC * NS
    nchunk = packed3.shape[1]
    epw = nchunk * CHUNK
    assert packed3.shape == (nw, nchunk, CHUNK) and epw * nw == w.shape[0]
    assert (nchunk - 1) % NBUF == 0
    row_blk = 80
    nrow_blk = n // row_blk
    assert nrow_blk * row_blk == n
    WOFF = 16

    mesh = plsc.VectorSubcoreMesh(core_axis_name="c", subcore_axis_name="s")

    @functools.partial(
        pl.kernel,
        out_type=jax.ShapeDtypeStruct((NC, n, d), jnp.float32),
        mesh=mesh,
        scratch_types=[
            pltpu.VMEM_SHARED((n, d), jnp.float32),
            pltpu.VMEM((nchunk, CHUNK), jnp.int32),
            pltpu.VMEM((NBUF, CHUNK), jnp.int32),
            pltpu.VMEM((NBUF, CHUNK), jnp.int32),
            pltpu.VMEM((epw + WOFF,), jnp.float32),
            pltpu.VMEM((NBUF, CHUNK, d), jnp.float32),
            [pltpu.SemaphoreType.DMA] * NBUF,
            [pltpu.SemaphoreType.DMA] * NBUF,
        ],
        compiler_params=pltpu.CompilerParams(needs_layout_passes=False),
    )
    def sc_kernel(x_hbm, packed_hbm, w_hbm, zeros_hbm, part_hbm,
                  agg_sh, packed_v, src_v, dst_v, w_v, rows_v, gsem, ssem):
        c = lax.axis_index("c")
        s = lax.axis_index("s")
        wid = c * NS + s

        @pl.loop(s, nrow_blk, step=NS)
        def zero_loop(b):
            r0 = pl.multiple_of(b * row_blk, 8)
            pltpu.sync_copy(zeros_hbm.at[pl.ds(r0, row_blk)],
                            agg_sh.at[pl.ds(r0, row_blk)])
        plsc.subcore_barrier()

        pltpu.sync_copy(packed_hbm.at[wid], packed_v)
        pltpu.sync_copy(w_hbm.at[pl.ds(wid * epw, epw)],
                        w_v.at[pl.ds(WOFF, epw)])

        def unpack_idx(g, b):
            for i in range(CHUNK // 16):
                sl = pl.ds(i * 16, 16)
                v = packed_v[g, sl]
                src_v[b, sl] = v & jnp.int32(0xFFFF)
                dst_v[b, sl] = lax.shift_right_logical(v, jnp.int32(16))

        def start_gather(g, b):
            pltpu.async_copy(x_hbm.at[src_v.at[b]], rows_v.at[b], gsem[b])

        def wait_gather(b):
            pltpu.make_async_copy(
                x_hbm.at[src_v.at[b]], rows_v.at[b], gsem[b]).wait()

        def wait_scatter(b):
            pltpu.make_async_copy(
                rows_v.at[b], agg_sh.at[dst_v.at[b]], ssem[b]).wait()

        def scale_rows(g, b):
            wbase = g * CHUNK + WOFF
            for ei in range(CHUNK):
                wb = plsc.load_gather(
                    w_v, [jnp.full((16,), ei, dtype=jnp.int32) + wbase])
                for r in range(d // 16):
                    sl = pl.ds(r * 16, 16)
                    rows_v[b, ei, sl] = rows_v[b, ei, sl] * wb

        def start_scatter(b):
            pltpu.async_copy(rows_v.at[b], agg_sh.at[dst_v.at[b]],
                             ssem[b], add=True)

        unpack_idx(0, 0)
        start_gather(0, 0)

        @pl.loop(0, nchunk - 1, step=NBUF)
        def edge_loop(j):
            for k in range(NBUF):
                g = j + k
                kn = (k + 1) % NBUF
                wait_gather(k)

                @pl.when(g >= 1)
                def _():
                    wait_scatter(kn)
                unpack_idx(g + 1, kn)
                start_gather(g + 1, kn)
                scale_rows(g, k)
                start_scatter(k)

        gt = nchunk - 1
        wait_gather(0)
        wait_scatter(1)
        scale_rows(gt, 0)
        start_scatter(0)
        wait_scatter(0)

        plsc.subcore_barrier()

        @pl.loop(s, nrow_blk, step=NS)
        def flush_loop(b):
            r0 = pl.multiple_of(b * row_blk, 8)
            pltpu.sync_copy(agg_sh.at[pl.ds(r0, row_blk)],
                            part_hbm.at[c, pl.ds(r0, row_blk)])

    return sc_kernel(x, packed3, w, zeros)


def _tc_matmul_relu(partials, W0, W1, W2):
    _, n, d = partials.shape
    block_rows = 1000

    def body(p_ref, w0_ref, w1_ref, w2_ref, o_ref):
        pblk = p_ref[0] + p_ref[1]
        w = w0_ref[...] + w1_ref[...] + w2_ref[...]
        acc = jnp.dot(pblk, w, preferred_element_type=jnp.float32)
        o_ref[...] = jnp.maximum(acc, 0.0)

    return pl.pallas_call(
        body,
        grid=(n // block_rows,),
        in_specs=[
            pl.BlockSpec((2, block_rows, d), lambda i: (0, i, 0)),
            pl.BlockSpec((d, d), lambda i: (0, 0)),
            pl.BlockSpec((d, d), lambda i: (0, 0)),
            pl.BlockSpec((d, d), lambda i: (0, 0)),
        ],
        out_specs=pl.BlockSpec((block_rows, d), lambda i: (i, 0)),
        out_shape=jax.ShapeDtypeStruct((n, d), jnp.float32),
    )(partials, W0, W1, W2)


def kernel(x, edge_index, edge_weight, num_nodes, W0, W1, W2):
    e = edge_index.shape[1]
    nw = NC * NS
    nchunk = e // (nw * CHUNK)
    packed = jnp.bitwise_or(jnp.left_shift(edge_index[1], 16), edge_index[0])
    packed3 = packed.reshape(nw, nchunk, CHUNK)
    zeros = jnp.zeros(x.shape, dtype=jnp.float32)
    partials = _sc_segment_sum(x, packed3, edge_weight, zeros)
    return _tc_matmul_relu(partials, W0, W1, W2)
